# baseline (speedup 1.0000x reference)
import functools

import jax
import jax.numpy as jnp
from jax import lax
from jax.experimental import pallas as pl
from jax.experimental.pallas import tpu as pltpu
from jax.experimental.pallas import tpu_sc as plsc

DIM = 32
BATCH = 16384
FIELDS = 26

NC = 2
NS = 16
NW = NC * NS
G = 128
TCD = BATCH // G
TPW = TCD // NW
BPW = TPW * FIELDS
NBUF = TPW
RPW = TPW * G


def _sc_gather(x, table):
    mesh = plsc.VectorSubcoreMesh(core_axis_name="c", subcore_axis_name="s")

    scratch = (
        [pltpu.VMEM((RPW, FIELDS), jnp.int32),
         pltpu.VMEM((FIELDS * RPW,), jnp.int32)]
        + [pltpu.VMEM((G, DIM), jnp.float32) for _ in range(NBUF)]
        + [pltpu.VMEM((G * DIM,), jnp.float32) for _ in range(NBUF)]
        + [pltpu.SemaphoreType.DMA] * (2 * NBUF)
    )

    @functools.partial(
        pl.kernel,
        mesh=mesh,
        out_type=jax.ShapeDtypeStruct((FIELDS * DIM * BATCH,), jnp.float32),
        scratch_types=scratch,
        compiler_params=pltpu.CompilerParams(
            use_tc_tiling_on_sc=False, needs_layout_passes=False),
    )
    def k(x_hbm, table_hbm, out_hbm, x_v, idxall, *bufs):
        rows = bufs[:NBUF]
        rowsT = bufs[NBUF:2 * NBUF]
        sem_g = bufs[2 * NBUF:3 * NBUF]
        sem_w = bufs[3 * NBUF:4 * NBUF]

        wid = lax.axis_index("s") * NC + lax.axis_index("c")
        tc0 = wid * TPW

        pltpu.sync_copy(x_hbm.at[pl.ds(wid * RPW, RPW)], x_v)

        iota = lax.iota(jnp.int32, 16)
        iv128 = [(iota + h * 16) * G for h in range(2)]
        ivf = [iota * RPW, (iota + FIELDS - 16) * RPW]

        @pl.loop(0, RPW)
        def _(j):
            jsplat = jnp.full((16,), j, jnp.int32)
            plsc.store_scatter(idxall, [ivf[0] + jsplat],
                               x_v[j, pl.ds(0, 16)])
            plsc.store_scatter(idxall, [ivf[1] + jsplat],
                               x_v[j, pl.ds(FIELDS - 16, 16)])

        def gather(k_, u):
            f = k_ // TPW
            return pltpu.make_async_copy(
                table_hbm.at[idxall.at[pl.ds(f * RPW + u * G, G)]],
                rows[u], sem_g[u])

        def write(k_, u):
            f = k_ // TPW
            base = f * (DIM * BATCH) + (tc0 + u) * (8 * G)
            cps = []
            for tr in range(DIM // 8):
                cps.append(pltpu.make_async_copy(
                    rowsT[u].at[pl.ds(tr * (8 * G), 8 * G)],
                    out_hbm.at[pl.ds(base + tr * (8 * G * TCD), 8 * G)],
                    sem_w[u]))
            return cps

        def start_writes(k_, u):
            for cp in write(k_, u):
                cp.start()

        def wait_writes(k_, u):
            for cp in write(k_, u):
                cp.wait()

        def transpose(u):
            rows_u = rows[u]
            rowsT_u = rowsT[u]

            @pl.loop(0, G)
            def _(c):
                csplat = jnp.full((16,), c, jnp.int32)
                for h in range(2):
                    v = rows_u[c, pl.ds(h * 16, 16)]
                    plsc.store_scatter(rowsT_u, [iv128[h] + csplat], v)

        for u in range(NBUF):
            gather(u, u).start()

        for u in range(NBUF):
            gather(u, u).wait()
            transpose(u)
            gather(u + NBUF, u).start()
            start_writes(u, u)

        @pl.loop(NBUF, BPW - NBUF, step=NBUF)
        def _(i):
            for u in range(NBUF):
                k_ = i + u
                gather(k_, u).wait()
                wait_writes(k_ - NBUF, u)
                transpose(u)
                gather(k_ + NBUF, u).start()
                start_writes(k_, u)

        for u in range(NBUF):
            k_ = BPW - NBUF + u
            gather(k_, u).wait()
            wait_writes(k_ - NBUF, u)
            transpose(u)
            start_writes(k_, u)
        for u in range(NBUF):
            wait_writes(BPW - NBUF + u, u)

    return k(x, table)


def kernel(x, embedding):
    flat = _sc_gather(x.astype(jnp.int32), embedding)
    out5d = flat.reshape(FIELDS, DIM // 8, TCD, 8, G)
    return out5d.transpose(2, 4, 0, 1, 3).reshape(BATCH, FIELDS, DIM)


# speedup vs baseline: 1.0176x; 1.0176x over previous
import functools

import jax
import jax.numpy as jnp
from jax import lax
from jax.experimental import pallas as pl
from jax.experimental.pallas import tpu as pltpu
from jax.experimental.pallas import tpu_sc as plsc

DIM = 32
BATCH = 16384
FIELDS = 26

NC = 2
NS = 16
NW = NC * NS
G = 128
TCD = BATCH // G
TPW = TCD // NW
BPW = TPW * FIELDS
NBUF = TPW
RPW = TPW * G


def _sc_gather(xt, table):
    mesh = plsc.VectorSubcoreMesh(core_axis_name="c", subcore_axis_name="s")

    scratch = (
        [pltpu.VMEM((FIELDS * RPW,), jnp.int32)]
        + [pltpu.VMEM((G, DIM), jnp.float32) for _ in range(NBUF)]
        + [pltpu.VMEM((G * DIM,), jnp.float32) for _ in range(NBUF)]
        + [pltpu.SemaphoreType.DMA] * (2 * NBUF + 1)
    )

    @functools.partial(
        pl.kernel,
        mesh=mesh,
        out_type=jax.ShapeDtypeStruct((FIELDS * DIM * BATCH,), jnp.float32),
        scratch_types=scratch,
        compiler_params=pltpu.CompilerParams(
            use_tc_tiling_on_sc=False, needs_layout_passes=False),
    )
    def k(xt_hbm, table_hbm, out_hbm, idx_v, *bufs):
        rows = bufs[:NBUF]
        rowsT = bufs[NBUF:2 * NBUF]
        sem_g = bufs[2 * NBUF:3 * NBUF]
        sem_w = bufs[3 * NBUF:4 * NBUF]
        sem_x = bufs[4 * NBUF]

        wid = lax.axis_index("s") * NC + lax.axis_index("c")
        tc0 = wid * TPW

        xcps = [
            pltpu.make_async_copy(
                xt_hbm.at[f, pl.ds(tc0 * G, RPW)],
                idx_v.at[pl.ds(f * RPW, RPW)], sem_x)
            for f in range(FIELDS)
        ]
        for cp in xcps:
            cp.start()
        for cp in xcps:
            cp.wait()

        iota = lax.iota(jnp.int32, 16)
        iv128 = [(iota + h * 16) * G for h in range(2)]

        def gather(k_, u):
            f = k_ // TPW
            return pltpu.make_async_copy(
                table_hbm.at[idx_v.at[pl.ds(f * RPW + u * G, G)]],
                rows[u], sem_g[u])

        def write(k_, u):
            f = k_ // TPW
            base = f * (DIM * BATCH) + (tc0 + u) * (8 * G)
            cps = []
            for tr in range(DIM // 8):
                cps.append(pltpu.make_async_copy(
                    rowsT[u].at[pl.ds(tr * (8 * G), 8 * G)],
                    out_hbm.at[pl.ds(base + tr * (8 * G * TCD), 8 * G)],
                    sem_w[u]))
            return cps

        def start_writes(k_, u):
            for cp in write(k_, u):
                cp.start()

        def wait_writes(k_, u):
            for cp in write(k_, u):
                cp.wait()

        def transpose(u):
            rows_u = rows[u]
            rowsT_u = rowsT[u]

            @pl.loop(0, G)
            def _(c):
                csplat = jnp.full((16,), c, jnp.int32)
                for h in range(2):
                    v = rows_u[c, pl.ds(h * 16, 16)]
                    plsc.store_scatter(rowsT_u, [iv128[h] + csplat], v)

        for u in range(NBUF):
            gather(u, u).start()

        for u in range(NBUF):
            gather(u, u).wait()
            transpose(u)
            gather(u + NBUF, u).start()
            start_writes(u, u)

        @pl.loop(NBUF, BPW - NBUF, step=NBUF)
        def _(i):
            for u in range(NBUF):
                k_ = i + u
                gather(k_, u).wait()
                wait_writes(k_ - NBUF, u)
                transpose(u)
                gather(k_ + NBUF, u).start()
                start_writes(k_, u)

        for u in range(NBUF):
            k_ = BPW - NBUF + u
            gather(k_, u).wait()
            wait_writes(k_ - NBUF, u)
            transpose(u)
            start_writes(k_, u)
        for u in range(NBUF):
            wait_writes(BPW - NBUF + u, u)

    return k(xt, table)


def kernel(x, embedding):
    xt = x.T.astype(jnp.int32)
    flat = _sc_gather(xt, embedding)
    out5d = flat.reshape(FIELDS, DIM // 8, TCD, 8, G)
    return out5d.transpose(2, 4, 0, 1, 3).reshape(BATCH, FIELDS, DIM)
